# Initial kernel scaffold; baseline (speedup 1.0000x reference)
import jax
import jax.numpy as jnp
from jax.experimental import pallas as pl


def kernel(prediction):
    raise NotImplementedError("write your pallas kernel here")


# speedup vs baseline: 3.2508x; 3.2508x over previous
import jax
import jax.numpy as jnp
from jax.experimental import pallas as pl
from jax.experimental.pallas import tpu as pltpu

_CONF = 0.95
_IOU = 0.45
_DETS = 300
_NEG = float("-inf")
_CHUNKS = 2


def _nms_body(pred_ref, s_ref, l_ref, b_ref, run_ref, offs_ref):
    predT = pred_ref[0]
    F = predT.shape[0]
    N = predT.shape[1]
    C = F - 5
    obj = predT[4:5, :]
    cls = predT[5:, :]
    scores = cls * obj
    cx = predT[0:1, :]
    cy = predT[1:2, :]
    w = predT[2:3, :]
    h = predT[3:4, :]
    x1 = cx - w * 0.5
    y1 = cy - h * 0.5
    x2 = cx + w * 0.5
    y2 = cy + h * 0.5
    valid = scores > _CONF
    rv = jnp.any(valid, axis=0, keepdims=True)
    mc = jnp.maximum(
        jnp.maximum(jnp.max(jnp.where(rv, x1, _NEG)),
                    jnp.max(jnp.where(rv, y1, _NEG))),
        jnp.maximum(jnp.max(jnp.where(rv, x2, _NEG)),
                    jnp.max(jnp.where(rv, y2, _NEG))))
    step = mc + 1.0
    sub = jax.lax.broadcasted_iota(jnp.int32, (C, N), 0)
    offs_ref[...] = sub.astype(jnp.float32) * step
    run_ref[...] = jnp.where(valid, scores, _NEG)
    col = jax.lax.broadcasted_iota(jnp.int32, (1, N), 1)
    H = N // _CHUNKS

    def body(i, carry):
        run = run_ref[...]
        colmax = jnp.max(run, axis=0, keepdims=True)
        m = jnp.max(colmax)
        r = jnp.min(jnp.where(colmax == m, col, N))
        cmask = col == r
        c = jnp.min(jnp.where((run == m) & cmask,
                              jax.lax.broadcasted_iota(jnp.int32, (C, N), 0),
                              C))
        x1r = jnp.max(jnp.where(cmask, x1, _NEG))
        y1r = jnp.max(jnp.where(cmask, y1, _NEG))
        x2r = jnp.max(jnp.where(cmask, x2, _NEG))
        y2r = jnp.max(jnp.where(cmask, y2, _NEG))
        off_i = c.astype(jnp.float32) * step
        x1i = x1r + off_i
        y1i = y1r + off_i
        x2i = x2r + off_i
        y2i = y2r + off_i
        ai = (x2i - x1i) * (y2i - y1i)
        for k in range(_CHUNKS):
            sl = pl.dslice(k * H, H)
            offs_k = offs_ref[:, sl]
            run_k = run_ref[:, sl]
            x1o = x1[:, k * H:(k + 1) * H] + offs_k
            y1o = y1[:, k * H:(k + 1) * H] + offs_k
            x2o = x2[:, k * H:(k + 1) * H] + offs_k
            y2o = y2[:, k * H:(k + 1) * H] + offs_k
            areas = (x2o - x1o) * (y2o - y1o)
            ww = jnp.maximum(jnp.minimum(x2i, x2o) - jnp.maximum(x1i, x1o), 0.0)
            hh = jnp.maximum(jnp.minimum(y2i, y2o) - jnp.maximum(y1i, y1o), 0.0)
            inter = ww * hh
            iou = inter / (ai + areas - inter)
            run_ref[:, sl] = jnp.where(iou <= _IOU, run_k, _NEG)
        pick = cmask & (jax.lax.broadcasted_iota(jnp.int32, (C, N), 0) == c)
        run_ref[...] = jnp.where(pick, _NEG, run_ref[...])
        ok = m > _NEG
        one = (1, 1, 1)
        s_ref[:, pl.dslice(i, 1), :] = jnp.where(ok, m, 0.0).reshape(one)
        l_ref[:, pl.dslice(i, 1), :] = jnp.where(ok, c, -1).reshape(one)
        for j, v in enumerate((x1r, y1r, x2r, y2r)):
            b_ref[:, pl.dslice(i, 1), pl.dslice(j, 1)] = (
                jnp.where(ok, v, 0.0).reshape(one))
        return carry

    jax.lax.fori_loop(0, _DETS, body, 0)


def kernel(prediction):
    B, N, F = prediction.shape
    C = F - 5
    predT = prediction.transpose(0, 2, 1)
    s, l, b = pl.pallas_call(
        _nms_body,
        grid=(B,),
        in_specs=[pl.BlockSpec((1, F, N), lambda i: (i, 0, 0))],
        out_specs=[
            pl.BlockSpec((1, _DETS, 1), lambda i: (i, 0, 0)),
            pl.BlockSpec((1, _DETS, 1), lambda i: (i, 0, 0)),
            pl.BlockSpec((1, _DETS, 4), lambda i: (i, 0, 0)),
        ],
        out_shape=[
            jax.ShapeDtypeStruct((B, _DETS, 1), jnp.float32),
            jax.ShapeDtypeStruct((B, _DETS, 1), jnp.int32),
            jax.ShapeDtypeStruct((B, _DETS, 4), jnp.float32),
        ],
        scratch_shapes=[pltpu.VMEM((C, N), jnp.float32),
                        pltpu.VMEM((C, N), jnp.float32)],
        compiler_params=pltpu.CompilerParams(
            dimension_semantics=("arbitrary",)),
    )(predT)
    return (s.reshape(B, _DETS), l.reshape(B, _DETS), b)


# speedup vs baseline: 10.3122x; 3.1722x over previous
import jax
import jax.numpy as jnp
from jax.experimental import pallas as pl
from jax.experimental.pallas import tpu as pltpu

_CONF = 0.95
_IOU = 0.45
_DETS = 300
_NEG = float("-inf")
_CHUNKS = 2


def _nms_body(pred_ref, s_ref, l_ref, b_ref, run_ref):
    predT = pred_ref[0]
    F = predT.shape[0]
    N = predT.shape[1]
    C = F - 5
    obj = predT[4:5, :]
    cls = predT[5:, :]
    scores = cls * obj
    cx = predT[0:1, :]
    cy = predT[1:2, :]
    w = predT[2:3, :]
    h = predT[3:4, :]
    x1 = cx - w * 0.5
    y1 = cy - h * 0.5
    x2 = cx + w * 0.5
    y2 = cy + h * 0.5
    valid = scores > _CONF
    rv = jnp.any(valid, axis=0, keepdims=True)
    mc = jnp.maximum(
        jnp.maximum(jnp.max(jnp.where(rv, x1, _NEG)),
                    jnp.max(jnp.where(rv, y1, _NEG))),
        jnp.maximum(jnp.max(jnp.where(rv, x2, _NEG)),
                    jnp.max(jnp.where(rv, y2, _NEG))))
    step = mc + 1.0
    run_ref[...] = jnp.where(valid, scores, _NEG)
    col = jax.lax.broadcasted_iota(jnp.int32, (1, N), 1)

    def body(i, carry):
        run = run_ref[...]
        colmax = jnp.max(run, axis=0, keepdims=True)
        m = jnp.max(colmax)
        r = jnp.min(jnp.where(colmax == m, col, N))
        cmask = col == r
        c = jnp.min(jnp.where((run == m) & cmask,
                              jax.lax.broadcasted_iota(jnp.int32, (C, N), 0),
                              C))
        x1r = jnp.max(jnp.where(cmask, x1, _NEG))
        y1r = jnp.max(jnp.where(cmask, y1, _NEG))
        x2r = jnp.max(jnp.where(cmask, x2, _NEG))
        y2r = jnp.max(jnp.where(cmask, y2, _NEG))
        off_i = c.astype(jnp.float32) * step
        x1i = x1r + off_i
        y1i = y1r + off_i
        x2i = x2r + off_i
        y2i = y2r + off_i
        ai = (x2i - x1i) * (y2i - y1i)
        row = pl.dslice(c, 1)
        run_c = run_ref[row, :]
        x1o = x1 + off_i
        y1o = y1 + off_i
        x2o = x2 + off_i
        y2o = y2 + off_i
        areas = (x2o - x1o) * (y2o - y1o)
        ww = jnp.maximum(jnp.minimum(x2i, x2o) - jnp.maximum(x1i, x1o), 0.0)
        hh = jnp.maximum(jnp.minimum(y2i, y2o) - jnp.maximum(y1i, y1o), 0.0)
        inter = ww * hh
        iou = inter / (ai + areas - inter)
        run_ref[row, :] = jnp.where(iou <= _IOU, run_c, _NEG)
        ok = m > _NEG
        one = (1, 1, 1)
        s_ref[:, pl.dslice(i, 1), :] = jnp.where(ok, m, 0.0).reshape(one)
        l_ref[:, pl.dslice(i, 1), :] = jnp.where(ok, c, -1).reshape(one)
        for j, v in enumerate((x1r, y1r, x2r, y2r)):
            b_ref[:, pl.dslice(i, 1), pl.dslice(j, 1)] = (
                jnp.where(ok, v, 0.0).reshape(one))
        return carry

    jax.lax.fori_loop(0, _DETS, body, 0)


def kernel(prediction):
    B, N, F = prediction.shape
    C = F - 5
    predT = prediction.transpose(0, 2, 1)
    s, l, b = pl.pallas_call(
        _nms_body,
        grid=(B,),
        in_specs=[pl.BlockSpec((1, F, N), lambda i: (i, 0, 0))],
        out_specs=[
            pl.BlockSpec((1, _DETS, 1), lambda i: (i, 0, 0)),
            pl.BlockSpec((1, _DETS, 1), lambda i: (i, 0, 0)),
            pl.BlockSpec((1, _DETS, 4), lambda i: (i, 0, 0)),
        ],
        out_shape=[
            jax.ShapeDtypeStruct((B, _DETS, 1), jnp.float32),
            jax.ShapeDtypeStruct((B, _DETS, 1), jnp.int32),
            jax.ShapeDtypeStruct((B, _DETS, 4), jnp.float32),
        ],
        scratch_shapes=[pltpu.VMEM((C, N), jnp.float32)],
        compiler_params=pltpu.CompilerParams(
            dimension_semantics=("arbitrary",)),
    )(predT)
    return (s.reshape(B, _DETS), l.reshape(B, _DETS), b)
